# baseline (speedup 1.0000x reference)
import functools

import jax
import jax.numpy as jnp
from jax.experimental import pallas as pl
from jax.experimental.pallas import tpu as pltpu

_N = 10000
_TM = 400


def _feat_body(x_ref, w_ref, o_ref):
    o_ref[...] = jnp.dot(
        x_ref[...].astype(jnp.bfloat16),
        w_ref[...].astype(jnp.bfloat16),
        preferred_element_type=jnp.float32,
    )


def _feat_matmul(x, w):
    n, f = x.shape
    k = w.shape[1]
    tm = 2000
    return pl.pallas_call(
        _feat_body,
        grid=(n // tm,),
        in_specs=[
            pl.BlockSpec((tm, f), lambda i: (i, 0)),
            pl.BlockSpec((f, k), lambda i: (0, 0)),
        ],
        out_specs=pl.BlockSpec((tm, k), lambda i: (i, 0)),
        out_shape=jax.ShapeDtypeStruct((n, k), jnp.float32),
    )(x, w)


def _gc_body(a_ref, s_ref, b_ref, *rest, has_w):
    if has_w:
        w_ref, o_ref = rest
    else:
        (o_ref,) = rest
    acc = jnp.dot(
        a_ref[...].astype(jnp.bfloat16),
        s_ref[...].astype(jnp.bfloat16),
        preferred_element_type=jnp.float32,
    )
    h = jnp.maximum(acc + b_ref[...], 0.0)
    if has_w:
        o_ref[...] = jnp.dot(
            h.astype(jnp.bfloat16),
            w_ref[...].astype(jnp.bfloat16),
            preferred_element_type=jnp.float32,
        )
    else:
        o_ref[...] = h


def _gc_layer(a, s, b, w_next):
    n = a.shape[0]
    kin = s.shape[1]
    kout = w_next.shape[1] if w_next is not None else kin
    has_w = w_next is not None
    body = functools.partial(_gc_body, has_w=has_w)
    in_specs = [
        pl.BlockSpec((_TM, _N), lambda i: (i, 0)),
        pl.BlockSpec((_N, kin), lambda i: (0, 0)),
        pl.BlockSpec((1, kin), lambda i: (0, 0)),
    ]
    args = [a, s, b.reshape(1, kin)]
    if has_w:
        in_specs.append(pl.BlockSpec((kin, kout), lambda i: (0, 0)))
        args.append(w_next)
    return pl.pallas_call(
        body,
        grid=(n // _TM,),
        in_specs=in_specs,
        out_specs=pl.BlockSpec((_TM, kout), lambda i: (i, 0)),
        out_shape=jax.ShapeDtypeStruct((n, kout), jnp.float32),
        compiler_params=pltpu.CompilerParams(
            dimension_semantics=("arbitrary",),
        ),
    )(*args)


def kernel(x, adj, A2, W1, b1, W2, b2, W3, b3):
    s1 = _feat_matmul(x, W1)
    s2 = _gc_layer(adj, s1, b1, W2)
    s3 = _gc_layer(adj, s2, b2, W3)
    s4 = _gc_layer(adj, s3, b3, W1)
    s5 = _gc_layer(A2, s4, b1, W2)
    return _gc_layer(A2, s5, b2, None)
